# baseline (speedup 1.0000x reference)
import functools

import jax
import jax.numpy as jnp
from jax import lax
from jax.experimental import pallas as pl
from jax.experimental.pallas import tpu as pltpu
from jax.experimental.pallas import tpu_sc as plsc


def _stage1_body(x1_ref, x2t_ref, p2_ref, W1_ref,
                 gidx_ref, w_ref, q2_ref, *, N2, C2, RB):
    b = pl.program_id(0)
    j = pl.program_id(1)

    @pl.when(j == 0)
    def _():
        q2_ref[...] = jnp.dot(p2_ref[...], W1_ref[:C2, :],
                              preferred_element_type=jnp.float32)

    x1 = x1_ref[...]
    x2t = x2t_ref[...]

    d0 = x1[:, 0:1] - x2t[0:1, :]
    d1c = x1[:, 1:2] - x2t[1:2, :]
    d2c = x1[:, 2:3] - x2t[2:3, :]
    d = (d0 * d0 + d1c * d1c) + d2c * d2c

    inf = jnp.float32(jnp.inf)
    m1 = jnp.min(d, axis=1, keepdims=True)
    eq1 = d == m1
    d1 = jnp.where(eq1, inf, d)
    m2 = jnp.min(d1, axis=1, keepdims=True)
    eq2 = d1 == m2
    d2 = jnp.where(eq2, inf, d1)
    m3 = jnp.min(d2, axis=1, keepdims=True)
    eq3 = d2 == m3

    r = [1.0 / jnp.maximum(m, 1e-10) for m in (m1, m2, m3)]
    norm = (r[0] + r[1]) + r[2]

    iota = lax.broadcasted_iota(jnp.int32, (RB, N2), 1)
    base = b * N2
    for k, eq in enumerate((eq1, eq2, eq3)):
        idx_k = jnp.min(jnp.where(eq, iota, N2), axis=1, keepdims=True)
        gidx_ref[:, k:k + 1] = idx_k + base
        w_ref[:, k:k + 1] = r[k] / norm


def _sc_gather_body(gidx_hbm, wf_hbm, q2_hbm, out_hbm,
                    idx_v, w_v, rows_v, out_v, sem,
                    *, PPW, CHUNK, H, NC):
    wid = lax.axis_index("s") * NC + lax.axis_index("c")
    base = wid * PPW
    nchunks = PPW // CHUNK

    def chunk_body(ci, carry):
        pbase = base + ci * CHUNK
        cbase3 = pbase * 3
        pltpu.sync_copy(gidx_hbm.at[pl.ds(cbase3, CHUNK * 3)], idx_v)
        pltpu.sync_copy(wf_hbm.at[pl.ds(cbase3, CHUNK * 3)],
                        w_v.at[pl.ds(0, CHUNK * 3)])
        pltpu.async_copy(q2_hbm.at[idx_v], rows_v, sem).wait()

        def point_body(p, c2):
            p3 = p * 3
            wtri = w_v[pl.ds(p3, 16)]
            zz = jnp.zeros((16,), jnp.int32)
            dn = lax.GatherDimensionNumbers(offset_dims=(),
                                            collapsed_slice_dims=(0,),
                                            start_index_map=(0,))
            splat = lambda i: lax.gather(
                wtri, (zz + i)[:, None], dn, slice_sizes=(1,),
                mode=lax.GatherScatterMode.PROMISE_IN_BOUNDS)
            w0, w1, w2 = splat(0), splat(1), splat(2)
            for c in range(H // 16):
                sl = pl.ds(c * 16, 16)
                out_v[p, sl] = (w0 * rows_v[p3, sl]
                                + w1 * rows_v[p3 + 1, sl]
                                + w2 * rows_v[p3 + 2, sl])
            return c2

        lax.fori_loop(0, CHUNK, point_body, 0)
        pltpu.sync_copy(out_v, out_hbm.at[pl.ds(pbase, CHUNK)])
        return carry

    lax.fori_loop(0, nchunks, chunk_body, 0)


def _stage2_body(interp_ref, p1_ref, W1_ref, b1_ref, W2_ref, b2_ref,
                 out_ref, *, C2):
    p1h = jnp.dot(p1_ref[...], W1_ref[C2:, :],
                  preferred_element_type=jnp.float32)
    h1 = jnp.maximum(interp_ref[...] + p1h + b1_ref[...], 0.0)
    h2 = jnp.dot(h1, W2_ref[...], preferred_element_type=jnp.float32)
    out_ref[...] = jnp.maximum(h2 + b2_ref[...], 0.0)


def kernel(xyz1, xyz2, points1, points2, W1, b1, W2, b2):
    B, N1, _ = xyz1.shape
    N2 = xyz2.shape[1]
    C1 = points1.shape[2]
    C2 = points2.shape[2]
    H = W1.shape[1]
    O = W2.shape[1]
    P = B * N1

    RB = 1024
    NB = N1 // RB

    x2t = jnp.transpose(xyz2, (0, 2, 1))
    b1r = b1.reshape(1, H)
    b2r = b2.reshape(1, O)

    s1 = functools.partial(_stage1_body, N2=N2, C2=C2, RB=RB)
    gidx, wf, q2 = pl.pallas_call(
        s1,
        grid=(B, NB),
        in_specs=[
            pl.BlockSpec((None, RB, 3), lambda b, j: (b, j, 0)),
            pl.BlockSpec((None, 3, N2), lambda b, j: (b, 0, 0)),
            pl.BlockSpec((None, N2, C2), lambda b, j: (b, 0, 0)),
            pl.BlockSpec((C1 + C2, H), lambda b, j: (0, 0)),
        ],
        out_specs=[
            pl.BlockSpec((None, RB, 3), lambda b, j: (b, j, 0)),
            pl.BlockSpec((None, RB, 3), lambda b, j: (b, j, 0)),
            pl.BlockSpec((None, N2, H), lambda b, j: (b, 0, 0)),
        ],
        out_shape=[
            jax.ShapeDtypeStruct((B, N1, 3), jnp.int32),
            jax.ShapeDtypeStruct((B, N1, 3), jnp.float32),
            jax.ShapeDtypeStruct((B, N2, H), jnp.float32),
        ],
    )(xyz1, x2t, points2, W1)

    gidxf = gidx.reshape(P * 3)
    wff = wf.reshape(P * 3)
    q2f = q2.reshape(B * N2, H)

    info = plsc.get_sparse_core_info()
    NC, NS = info.num_cores, info.num_subcores
    NW = NC * NS
    PPW = P // NW
    CHUNK = 64

    mesh = plsc.VectorSubcoreMesh(core_axis_name="c", subcore_axis_name="s")
    sc_body = functools.partial(_sc_gather_body, PPW=PPW, CHUNK=CHUNK,
                                H=H, NC=NC)
    interp = pl.kernel(
        sc_body,
        mesh=mesh,
        out_type=jax.ShapeDtypeStruct((P, H), jnp.float32),
        scratch_types=[
            pltpu.VMEM((CHUNK * 3,), jnp.int32),
            pltpu.VMEM((CHUNK * 3 + 16,), jnp.float32),
            pltpu.VMEM((CHUNK * 3, H), jnp.float32),
            pltpu.VMEM((CHUNK, H), jnp.float32),
            pltpu.SemaphoreType.DMA,
        ],
    )(gidxf, wff, q2f)

    RB2 = 2048
    s2 = functools.partial(_stage2_body, C2=C2)
    out = pl.pallas_call(
        s2,
        grid=(P // RB2,),
        in_specs=[
            pl.BlockSpec((RB2, H), lambda i: (i, 0)),
            pl.BlockSpec((RB2, C1), lambda i: (i, 0)),
            pl.BlockSpec((C1 + C2, H), lambda i: (0, 0)),
            pl.BlockSpec((1, H), lambda i: (0, 0)),
            pl.BlockSpec((H, O), lambda i: (0, 0)),
            pl.BlockSpec((1, O), lambda i: (0, 0)),
        ],
        out_specs=pl.BlockSpec((RB2, O), lambda i: (i, 0)),
        out_shape=jax.ShapeDtypeStruct((P, O), jnp.float32),
    )(interp, points1.reshape(P, C1), W1, b1r, W2, b2r)
    return out.reshape(B, N1, O)


# speedup vs baseline: 1.1384x; 1.1384x over previous
import functools

import jax
import jax.numpy as jnp
from jax import lax
from jax.experimental import pallas as pl
from jax.experimental.pallas import tpu as pltpu
from jax.experimental.pallas import tpu_sc as plsc


def _stage1_body(x1_ref, x2t_ref, p2_ref, W1_ref,
                 gidx_ref, w_ref, q2_ref, *, N2, C2, RB):
    b = pl.program_id(0)
    j = pl.program_id(1)

    @pl.when(j == 0)
    def _():
        q2_ref[...] = jnp.dot(p2_ref[...], W1_ref[:C2, :],
                              preferred_element_type=jnp.float32)

    x1 = x1_ref[...]
    x2t = x2t_ref[...]

    d0 = x1[:, 0:1] - x2t[0:1, :]
    d1c = x1[:, 1:2] - x2t[1:2, :]
    d2c = x1[:, 2:3] - x2t[2:3, :]
    d = (d0 * d0 + d1c * d1c) + d2c * d2c

    inf = jnp.float32(jnp.inf)
    m1 = jnp.min(d, axis=1, keepdims=True)
    eq1 = d == m1
    d1 = jnp.where(eq1, inf, d)
    m2 = jnp.min(d1, axis=1, keepdims=True)
    eq2 = d1 == m2
    d2 = jnp.where(eq2, inf, d1)
    m3 = jnp.min(d2, axis=1, keepdims=True)
    eq3 = d2 == m3

    r = [1.0 / jnp.maximum(m, 1e-10) for m in (m1, m2, m3)]
    norm = (r[0] + r[1]) + r[2]

    iota = lax.broadcasted_iota(jnp.int32, (RB, N2), 1)
    base = b * N2
    for k, eq in enumerate((eq1, eq2, eq3)):
        idx_k = jnp.min(jnp.where(eq, iota, N2), axis=1, keepdims=True)
        gidx_ref[:, k:k + 1] = idx_k + base
        w_ref[:, k:k + 1] = r[k] / norm


def _sc_gather_body(gidx_hbm, wf_hbm, q2_hbm, out_hbm,
                    idx_all, w_all, rows0, rows1, out_v, sem0, sem1,
                    *, PPW, CHUNK, H, NC):
    wid = lax.axis_index("s") * NC + lax.axis_index("c")
    base = wid * PPW
    nchunks = PPW // CHUNK
    npairs = nchunks // 2

    pltpu.sync_copy(gidx_hbm.at[pl.ds(base * 3, PPW * 3)], idx_all)
    pltpu.sync_copy(wf_hbm.at[pl.ds(base * 3, PPW * 3)],
                    w_all.at[pl.ds(0, PPW * 3)])

    def start_gather(ci, rows_v, sem):
        idx_sl = idx_all.at[pl.ds(ci * CHUNK * 3, CHUNK * 3)]
        return pltpu.async_copy(q2_hbm.at[idx_sl], rows_v, sem)

    def wait_gather(ci, rows_v, sem):
        idx_sl = idx_all.at[pl.ds(ci * CHUNK * 3, CHUNK * 3)]
        pltpu.make_async_copy(q2_hbm.at[idx_sl], rows_v, sem).wait()

    def compute_chunk(ci, rows_v):
        wbase3 = ci * CHUNK * 3
        zz = jnp.zeros((16,), jnp.int32)
        dn = lax.GatherDimensionNumbers(offset_dims=(),
                                        collapsed_slice_dims=(0,),
                                        start_index_map=(0,))

        def grp_body(g, c2):
            for u in range(4):
                p = g * 4 + u
                p3 = p * 3
                wtri = w_all[pl.ds(wbase3 + p3, 16)]
                splat = lambda i: lax.gather(
                    wtri, (zz + i)[:, None], dn, slice_sizes=(1,),
                    mode=lax.GatherScatterMode.PROMISE_IN_BOUNDS)
                w0, w1, w2 = splat(0), splat(1), splat(2)
                for c in range(H // 16):
                    sl = pl.ds(c * 16, 16)
                    out_v[p, sl] = (w0 * rows_v[p3, sl]
                                    + w1 * rows_v[p3 + 1, sl]
                                    + w2 * rows_v[p3 + 2, sl])
            return c2

        lax.fori_loop(0, CHUNK // 4, grp_body, 0)
        pltpu.sync_copy(out_v, out_hbm.at[pl.ds(base + ci * CHUNK, CHUNK)])

    start_gather(0, rows0, sem0)

    def pair_body(q, carry):
        c0 = 2 * q
        start_gather(c0 + 1, rows1, sem1)
        wait_gather(c0, rows0, sem0)
        compute_chunk(c0, rows0)

        @pl.when(q + 1 < npairs)
        def _():
            start_gather(c0 + 2, rows0, sem0)

        wait_gather(c0 + 1, rows1, sem1)
        compute_chunk(c0 + 1, rows1)
        return carry

    lax.fori_loop(0, npairs, pair_body, 0)


def _stage2_body(interp_ref, p1_ref, W1_ref, b1_ref, W2_ref, b2_ref,
                 out_ref, *, C2):
    p1h = jnp.dot(p1_ref[...], W1_ref[C2:, :],
                  preferred_element_type=jnp.float32)
    h1 = jnp.maximum(interp_ref[...] + p1h + b1_ref[...], 0.0)
    h2 = jnp.dot(h1, W2_ref[...], preferred_element_type=jnp.float32)
    out_ref[...] = jnp.maximum(h2 + b2_ref[...], 0.0)


def kernel(xyz1, xyz2, points1, points2, W1, b1, W2, b2):
    B, N1, _ = xyz1.shape
    N2 = xyz2.shape[1]
    C1 = points1.shape[2]
    C2 = points2.shape[2]
    H = W1.shape[1]
    O = W2.shape[1]
    P = B * N1

    RB = 1024
    NB = N1 // RB

    x2t = jnp.transpose(xyz2, (0, 2, 1))
    b1r = b1.reshape(1, H)
    b2r = b2.reshape(1, O)

    s1 = functools.partial(_stage1_body, N2=N2, C2=C2, RB=RB)
    gidx, wf, q2 = pl.pallas_call(
        s1,
        grid=(B, NB),
        in_specs=[
            pl.BlockSpec((None, RB, 3), lambda b, j: (b, j, 0)),
            pl.BlockSpec((None, 3, N2), lambda b, j: (b, 0, 0)),
            pl.BlockSpec((None, N2, C2), lambda b, j: (b, 0, 0)),
            pl.BlockSpec((C1 + C2, H), lambda b, j: (0, 0)),
        ],
        out_specs=[
            pl.BlockSpec((None, RB, 3), lambda b, j: (b, j, 0)),
            pl.BlockSpec((None, RB, 3), lambda b, j: (b, j, 0)),
            pl.BlockSpec((None, N2, H), lambda b, j: (b, 0, 0)),
        ],
        out_shape=[
            jax.ShapeDtypeStruct((B, N1, 3), jnp.int32),
            jax.ShapeDtypeStruct((B, N1, 3), jnp.float32),
            jax.ShapeDtypeStruct((B, N2, H), jnp.float32),
        ],
    )(xyz1, x2t, points2, W1)

    gidxf = gidx.reshape(P * 3)
    wff = wf.reshape(P * 3)
    q2f = q2.reshape(B * N2, H)

    info = plsc.get_sparse_core_info()
    NC, NS = info.num_cores, info.num_subcores
    NW = NC * NS
    PPW = P // NW
    CHUNK = 64

    mesh = plsc.VectorSubcoreMesh(core_axis_name="c", subcore_axis_name="s")
    sc_body = functools.partial(_sc_gather_body, PPW=PPW, CHUNK=CHUNK,
                                H=H, NC=NC)
    interp = pl.kernel(
        sc_body,
        mesh=mesh,
        out_type=jax.ShapeDtypeStruct((P, H), jnp.float32),
        scratch_types=[
            pltpu.VMEM((PPW * 3,), jnp.int32),
            pltpu.VMEM((PPW * 3 + 16,), jnp.float32),
            pltpu.VMEM((CHUNK * 3, H), jnp.float32),
            pltpu.VMEM((CHUNK * 3, H), jnp.float32),
            pltpu.VMEM((CHUNK, H), jnp.float32),
            pltpu.SemaphoreType.DMA,
            pltpu.SemaphoreType.DMA,
        ],
    )(gidxf, wff, q2f)

    RB2 = 2048
    s2 = functools.partial(_stage2_body, C2=C2)
    out = pl.pallas_call(
        s2,
        grid=(P // RB2,),
        in_specs=[
            pl.BlockSpec((RB2, H), lambda i: (i, 0)),
            pl.BlockSpec((RB2, C1), lambda i: (i, 0)),
            pl.BlockSpec((C1 + C2, H), lambda i: (0, 0)),
            pl.BlockSpec((1, H), lambda i: (0, 0)),
            pl.BlockSpec((H, O), lambda i: (0, 0)),
            pl.BlockSpec((1, O), lambda i: (0, 0)),
        ],
        out_specs=pl.BlockSpec((RB2, O), lambda i: (i, 0)),
        out_shape=jax.ShapeDtypeStruct((P, O), jnp.float32),
    )(interp, points1.reshape(P, C1), W1, b1r, W2, b2r)
    return out.reshape(B, N1, O)


# speedup vs baseline: 1.2240x; 1.0752x over previous
import functools

import jax
import jax.numpy as jnp
from jax import lax
from jax.experimental import pallas as pl
from jax.experimental.pallas import tpu as pltpu
from jax.experimental.pallas import tpu_sc as plsc


def _stage1_body(x1_ref, x2t_ref, p2_ref, W1_ref,
                 gidx_ref, w_ref, q2_ref, *, N2, C2, RB):
    b = pl.program_id(0)
    j = pl.program_id(1)

    @pl.when(j == 0)
    def _():
        q2_ref[...] = jnp.dot(p2_ref[...], W1_ref[:C2, :],
                              preferred_element_type=jnp.float32)

    x1 = x1_ref[...]
    x2t = x2t_ref[...]

    d0 = x1[:, 0:1] - x2t[0:1, :]
    d1c = x1[:, 1:2] - x2t[1:2, :]
    d2c = x1[:, 2:3] - x2t[2:3, :]
    d = (d0 * d0 + d1c * d1c) + d2c * d2c

    inf = jnp.float32(jnp.inf)
    m1 = jnp.min(d, axis=1, keepdims=True)
    eq1 = d == m1
    d1 = jnp.where(eq1, inf, d)
    m2 = jnp.min(d1, axis=1, keepdims=True)
    eq2 = d1 == m2
    d2 = jnp.where(eq2, inf, d1)
    m3 = jnp.min(d2, axis=1, keepdims=True)
    eq3 = d2 == m3

    r = [1.0 / jnp.maximum(m, 1e-10) for m in (m1, m2, m3)]
    norm = (r[0] + r[1]) + r[2]

    iota = lax.broadcasted_iota(jnp.int32, (RB, N2), 1)
    base = b * N2
    for k, eq in enumerate((eq1, eq2, eq3)):
        idx_k = jnp.min(jnp.where(eq, iota, N2), axis=1, keepdims=True)
        gidx_ref[:, k:k + 1] = idx_k + base
        w_ref[:, k:k + 1] = r[k] / norm


def _sc_gather_body(gidx_hbm, wf_hbm, q2_hbm, out_hbm,
                    idx_all, w_all, rows0, rows1, out_v, sem0, sem1,
                    *, PPW, CHUNK, H, NC):
    wid = lax.axis_index("s") * NC + lax.axis_index("c")
    base = wid * PPW
    nchunks = PPW // CHUNK
    npairs = nchunks // 2

    pltpu.sync_copy(gidx_hbm.at[pl.ds(base * 3, PPW * 3)], idx_all)
    pltpu.sync_copy(wf_hbm.at[pl.ds(base * 3, PPW * 3)],
                    w_all.at[pl.ds(0, PPW * 3)])

    def start_gather(ci, rows_v, sem):
        idx_sl = idx_all.at[pl.ds(ci * CHUNK * 3, CHUNK * 3)]
        return pltpu.async_copy(q2_hbm.at[idx_sl], rows_v, sem)

    def wait_gather(ci, rows_v, sem):
        idx_sl = idx_all.at[pl.ds(ci * CHUNK * 3, CHUNK * 3)]
        pltpu.make_async_copy(q2_hbm.at[idx_sl], rows_v, sem).wait()

    def compute_chunk(ci, rows_v):
        wbase3 = ci * CHUNK * 3
        zz = jnp.zeros((16,), jnp.int32)
        dn = lax.GatherDimensionNumbers(offset_dims=(),
                                        collapsed_slice_dims=(0,),
                                        start_index_map=(0,))

        def grp_body(g, c2):
            for u in range(4):
                p = g * 4 + u
                p3 = p * 3
                wtri = w_all[pl.ds(wbase3 + p3, 16)]
                splat = lambda i: lax.gather(
                    wtri, (zz + i)[:, None], dn, slice_sizes=(1,),
                    mode=lax.GatherScatterMode.PROMISE_IN_BOUNDS)
                w0, w1, w2 = splat(0), splat(1), splat(2)
                for c in range(H // 16):
                    sl = pl.ds(c * 16, 16)
                    out_v[p, sl] = (w0 * rows_v[p3, sl]
                                    + w1 * rows_v[p3 + 1, sl]
                                    + w2 * rows_v[p3 + 2, sl])
            return c2

        lax.fori_loop(0, CHUNK // 4, grp_body, 0)
        pltpu.sync_copy(out_v, out_hbm.at[pl.ds(base + ci * CHUNK, CHUNK)])

    start_gather(0, rows0, sem0)

    def pair_body(q, carry):
        c0 = 2 * q
        start_gather(c0 + 1, rows1, sem1)
        wait_gather(c0, rows0, sem0)
        compute_chunk(c0, rows0)

        @pl.when(q + 1 < npairs)
        def _():
            start_gather(c0 + 2, rows0, sem0)

        wait_gather(c0 + 1, rows1, sem1)
        compute_chunk(c0 + 1, rows1)
        return carry

    lax.fori_loop(0, npairs, pair_body, 0)


def _stage2_body(interp_ref, p1_ref, W1_ref, b1_ref, W2_ref, b2_ref,
                 out_ref, *, C2):
    p1h = jnp.dot(p1_ref[...], W1_ref[C2:, :],
                  preferred_element_type=jnp.float32)
    h1 = jnp.maximum(interp_ref[...] + p1h + b1_ref[...], 0.0)
    h2 = jnp.dot(h1, W2_ref[...], preferred_element_type=jnp.float32)
    out_ref[...] = jnp.maximum(h2 + b2_ref[...], 0.0)


def kernel(xyz1, xyz2, points1, points2, W1, b1, W2, b2):
    B, N1, _ = xyz1.shape
    O = W2.shape[1]
    B2 = B // 2
    outs = [
        _half_pipeline(xyz1[h * B2:(h + 1) * B2],
                       xyz2[h * B2:(h + 1) * B2],
                       points1[h * B2:(h + 1) * B2],
                       points2[h * B2:(h + 1) * B2],
                       W1, b1, W2, b2)
        for h in range(2)
    ]
    return jnp.concatenate(outs, axis=0)


def _half_pipeline(xyz1, xyz2, points1, points2, W1, b1, W2, b2):
    B, N1, _ = xyz1.shape
    N2 = xyz2.shape[1]
    C1 = points1.shape[2]
    C2 = points2.shape[2]
    H = W1.shape[1]
    O = W2.shape[1]
    P = B * N1

    RB = 1024
    NB = N1 // RB

    x2t = jnp.transpose(xyz2, (0, 2, 1))
    b1r = b1.reshape(1, H)
    b2r = b2.reshape(1, O)

    s1 = functools.partial(_stage1_body, N2=N2, C2=C2, RB=RB)
    gidx, wf, q2 = pl.pallas_call(
        s1,
        grid=(B, NB),
        in_specs=[
            pl.BlockSpec((None, RB, 3), lambda b, j: (b, j, 0)),
            pl.BlockSpec((None, 3, N2), lambda b, j: (b, 0, 0)),
            pl.BlockSpec((None, N2, C2), lambda b, j: (b, 0, 0)),
            pl.BlockSpec((C1 + C2, H), lambda b, j: (0, 0)),
        ],
        out_specs=[
            pl.BlockSpec((None, RB, 3), lambda b, j: (b, j, 0)),
            pl.BlockSpec((None, RB, 3), lambda b, j: (b, j, 0)),
            pl.BlockSpec((None, N2, H), lambda b, j: (b, 0, 0)),
        ],
        out_shape=[
            jax.ShapeDtypeStruct((B, N1, 3), jnp.int32),
            jax.ShapeDtypeStruct((B, N1, 3), jnp.float32),
            jax.ShapeDtypeStruct((B, N2, H), jnp.float32),
        ],
    )(xyz1, x2t, points2, W1)

    gidxf = gidx.reshape(P * 3)
    wff = wf.reshape(P * 3)
    q2f = q2.reshape(B * N2, H)

    info = plsc.get_sparse_core_info()
    NC, NS = info.num_cores, info.num_subcores
    NW = NC * NS
    PPW = P // NW
    CHUNK = 64

    mesh = plsc.VectorSubcoreMesh(core_axis_name="c", subcore_axis_name="s")
    sc_body = functools.partial(_sc_gather_body, PPW=PPW, CHUNK=CHUNK,
                                H=H, NC=NC)
    interp = pl.kernel(
        sc_body,
        mesh=mesh,
        out_type=jax.ShapeDtypeStruct((P, H), jnp.float32),
        scratch_types=[
            pltpu.VMEM((PPW * 3,), jnp.int32),
            pltpu.VMEM((PPW * 3 + 16,), jnp.float32),
            pltpu.VMEM((CHUNK * 3, H), jnp.float32),
            pltpu.VMEM((CHUNK * 3, H), jnp.float32),
            pltpu.VMEM((CHUNK, H), jnp.float32),
            pltpu.SemaphoreType.DMA,
            pltpu.SemaphoreType.DMA,
        ],
    )(gidxf, wff, q2f)

    RB2 = 2048
    s2 = functools.partial(_stage2_body, C2=C2)
    out = pl.pallas_call(
        s2,
        grid=(P // RB2,),
        in_specs=[
            pl.BlockSpec((RB2, H), lambda i: (i, 0)),
            pl.BlockSpec((RB2, C1), lambda i: (i, 0)),
            pl.BlockSpec((C1 + C2, H), lambda i: (0, 0)),
            pl.BlockSpec((1, H), lambda i: (0, 0)),
            pl.BlockSpec((H, O), lambda i: (0, 0)),
            pl.BlockSpec((1, O), lambda i: (0, 0)),
        ],
        out_specs=pl.BlockSpec((RB2, O), lambda i: (i, 0)),
        out_shape=jax.ShapeDtypeStruct((P, O), jnp.float32),
    )(interp, points1.reshape(P, C1), W1, b1r, W2, b2r)
    return out.reshape(B, N1, O)


# speedup vs baseline: 1.2360x; 1.0097x over previous
import functools

import jax
import jax.numpy as jnp
from jax import lax
from jax.experimental import pallas as pl
from jax.experimental.pallas import tpu as pltpu
from jax.experimental.pallas import tpu_sc as plsc


def _stage1_body(x1_ref, x2t_ref, p2_ref, W1_ref,
                 gidx_ref, w_ref, q2_ref, *, N2, C2, RB):
    b = pl.program_id(0)
    j = pl.program_id(1)

    @pl.when(j == 0)
    def _():
        q2_ref[...] = jnp.dot(p2_ref[...], W1_ref[:C2, :],
                              preferred_element_type=jnp.float32)

    x1 = x1_ref[...]
    x2t = x2t_ref[...]

    d0 = x1[:, 0:1] - x2t[0:1, :]
    d1c = x1[:, 1:2] - x2t[1:2, :]
    d2c = x1[:, 2:3] - x2t[2:3, :]
    d = (d0 * d0 + d1c * d1c) + d2c * d2c

    inf = jnp.float32(jnp.inf)
    m1 = jnp.min(d, axis=1, keepdims=True)
    eq1 = d == m1
    d1 = jnp.where(eq1, inf, d)
    m2 = jnp.min(d1, axis=1, keepdims=True)
    eq2 = d1 == m2
    d2 = jnp.where(eq2, inf, d1)
    m3 = jnp.min(d2, axis=1, keepdims=True)
    eq3 = d2 == m3

    r = [1.0 / jnp.maximum(m, 1e-10) for m in (m1, m2, m3)]
    norm = (r[0] + r[1]) + r[2]

    iota = lax.broadcasted_iota(jnp.int32, (RB, N2), 1)
    base = b * N2
    for k, eq in enumerate((eq1, eq2, eq3)):
        idx_k = jnp.min(jnp.where(eq, iota, N2), axis=1, keepdims=True)
        gidx_ref[:, k:k + 1] = idx_k + base
        w_ref[:, k * 16:(k + 1) * 16] = jnp.broadcast_to(r[k] / norm,
                                                         (RB, 16))


def _sc_gather_body(gidx_hbm, wf_hbm, q2_hbm, out_hbm,
                    idx_all, w_all, rows0, rows1, out_v, sem0, sem1,
                    *, PPW, CHUNK, H, NC):
    wid = lax.axis_index("s") * NC + lax.axis_index("c")
    base = wid * PPW
    nchunks = PPW // CHUNK
    npairs = nchunks // 2

    pltpu.sync_copy(gidx_hbm.at[pl.ds(base * 3, PPW * 3)], idx_all)
    pltpu.sync_copy(wf_hbm.at[pl.ds(base * 48, PPW * 48)], w_all)

    def start_gather(ci, rows_v, sem):
        idx_sl = idx_all.at[pl.ds(ci * CHUNK * 3, CHUNK * 3)]
        return pltpu.async_copy(q2_hbm.at[idx_sl], rows_v, sem)

    def wait_gather(ci, rows_v, sem):
        idx_sl = idx_all.at[pl.ds(ci * CHUNK * 3, CHUNK * 3)]
        pltpu.make_async_copy(q2_hbm.at[idx_sl], rows_v, sem).wait()

    def compute_chunk(ci, rows_v):
        wbase3 = ci * CHUNK * 3

        def grp_body(g, c2):
            for u in range(4):
                p = g * 4 + u
                p3 = p * 3
                wb = (wbase3 + p3) * 16
                w0 = w_all[pl.ds(wb, 16)]
                w1 = w_all[pl.ds(wb + 16, 16)]
                w2 = w_all[pl.ds(wb + 32, 16)]
                for c in range(H // 16):
                    sl = pl.ds(c * 16, 16)
                    out_v[p, sl] = (w0 * rows_v[p3, sl]
                                    + w1 * rows_v[p3 + 1, sl]
                                    + w2 * rows_v[p3 + 2, sl])
            return c2

        lax.fori_loop(0, CHUNK // 4, grp_body, 0)
        pltpu.sync_copy(out_v, out_hbm.at[pl.ds(base + ci * CHUNK, CHUNK)])

    start_gather(0, rows0, sem0)

    def pair_body(q, carry):
        c0 = 2 * q
        start_gather(c0 + 1, rows1, sem1)
        wait_gather(c0, rows0, sem0)
        compute_chunk(c0, rows0)

        @pl.when(q + 1 < npairs)
        def _():
            start_gather(c0 + 2, rows0, sem0)

        wait_gather(c0 + 1, rows1, sem1)
        compute_chunk(c0 + 1, rows1)
        return carry

    lax.fori_loop(0, npairs, pair_body, 0)


def _stage2_body(interp_ref, p1_ref, W1_ref, b1_ref, W2_ref, b2_ref,
                 out_ref, *, C2):
    p1h = jnp.dot(p1_ref[...], W1_ref[C2:, :],
                  preferred_element_type=jnp.float32)
    h1 = jnp.maximum(interp_ref[...] + p1h + b1_ref[...], 0.0)
    h2 = jnp.dot(h1, W2_ref[...], preferred_element_type=jnp.float32)
    out_ref[...] = jnp.maximum(h2 + b2_ref[...], 0.0)


def kernel(xyz1, xyz2, points1, points2, W1, b1, W2, b2):
    B, N1, _ = xyz1.shape
    O = W2.shape[1]
    B2 = B // 2
    outs = [
        _half_pipeline(xyz1[h * B2:(h + 1) * B2],
                       xyz2[h * B2:(h + 1) * B2],
                       points1[h * B2:(h + 1) * B2],
                       points2[h * B2:(h + 1) * B2],
                       W1, b1, W2, b2)
        for h in range(2)
    ]
    return jnp.concatenate(outs, axis=0)


def _half_pipeline(xyz1, xyz2, points1, points2, W1, b1, W2, b2):
    B, N1, _ = xyz1.shape
    N2 = xyz2.shape[1]
    C1 = points1.shape[2]
    C2 = points2.shape[2]
    H = W1.shape[1]
    O = W2.shape[1]
    P = B * N1

    RB = 1024
    NB = N1 // RB

    x2t = jnp.transpose(xyz2, (0, 2, 1))
    b1r = b1.reshape(1, H)
    b2r = b2.reshape(1, O)

    s1 = functools.partial(_stage1_body, N2=N2, C2=C2, RB=RB)
    gidx, wf, q2 = pl.pallas_call(
        s1,
        grid=(B, NB),
        in_specs=[
            pl.BlockSpec((None, RB, 3), lambda b, j: (b, j, 0)),
            pl.BlockSpec((None, 3, N2), lambda b, j: (b, 0, 0)),
            pl.BlockSpec((None, N2, C2), lambda b, j: (b, 0, 0)),
            pl.BlockSpec((C1 + C2, H), lambda b, j: (0, 0)),
        ],
        out_specs=[
            pl.BlockSpec((None, RB, 3), lambda b, j: (b, j, 0)),
            pl.BlockSpec((None, RB, 48), lambda b, j: (b, j, 0)),
            pl.BlockSpec((None, N2, H), lambda b, j: (b, 0, 0)),
        ],
        out_shape=[
            jax.ShapeDtypeStruct((B, N1, 3), jnp.int32),
            jax.ShapeDtypeStruct((B, N1, 48), jnp.float32),
            jax.ShapeDtypeStruct((B, N2, H), jnp.float32),
        ],
    )(xyz1, x2t, points2, W1)

    gidxf = gidx.reshape(P * 3)
    wff = wf.reshape(P * 48)
    q2f = q2.reshape(B * N2, H)

    info = plsc.get_sparse_core_info()
    NC, NS = info.num_cores, info.num_subcores
    NW = NC * NS
    PPW = P // NW
    CHUNK = 32

    mesh = plsc.VectorSubcoreMesh(core_axis_name="c", subcore_axis_name="s")
    sc_body = functools.partial(_sc_gather_body, PPW=PPW, CHUNK=CHUNK,
                                H=H, NC=NC)
    interp = pl.kernel(
        sc_body,
        mesh=mesh,
        out_type=jax.ShapeDtypeStruct((P, H), jnp.float32),
        scratch_types=[
            pltpu.VMEM((PPW * 3,), jnp.int32),
            pltpu.VMEM((PPW * 48,), jnp.float32),
            pltpu.VMEM((CHUNK * 3, H), jnp.float32),
            pltpu.VMEM((CHUNK * 3, H), jnp.float32),
            pltpu.VMEM((CHUNK, H), jnp.float32),
            pltpu.SemaphoreType.DMA,
            pltpu.SemaphoreType.DMA,
        ],
    )(gidxf, wff, q2f)

    RB2 = 2048
    s2 = functools.partial(_stage2_body, C2=C2)
    out = pl.pallas_call(
        s2,
        grid=(P // RB2,),
        in_specs=[
            pl.BlockSpec((RB2, H), lambda i: (i, 0)),
            pl.BlockSpec((RB2, C1), lambda i: (i, 0)),
            pl.BlockSpec((C1 + C2, H), lambda i: (0, 0)),
            pl.BlockSpec((1, H), lambda i: (0, 0)),
            pl.BlockSpec((H, O), lambda i: (0, 0)),
            pl.BlockSpec((1, O), lambda i: (0, 0)),
        ],
        out_specs=pl.BlockSpec((RB2, O), lambda i: (i, 0)),
        out_shape=jax.ShapeDtypeStruct((P, O), jnp.float32),
    )(interp, points1.reshape(P, C1), W1, b1r, W2, b2r)
    return out.reshape(B, N1, O)


# speedup vs baseline: 1.2959x; 1.0485x over previous
import functools

import jax
import jax.numpy as jnp
from jax import lax
from jax.experimental import pallas as pl
from jax.experimental.pallas import tpu as pltpu
from jax.experimental.pallas import tpu_sc as plsc


def _stage1_body(x1_ref, x2t_ref, p2_ref, W1_ref,
                 gidx_ref, w_ref, q2_ref, *, N2, C2, RB):
    b = pl.program_id(0)
    j = pl.program_id(1)

    @pl.when(j == 0)
    def _():
        q2_ref[...] = jnp.dot(p2_ref[...], W1_ref[:C2, :],
                              preferred_element_type=jnp.float32)

    x1 = x1_ref[...]
    x2t = x2t_ref[...]

    d0 = x1[:, 0:1] - x2t[0:1, :]
    d1c = x1[:, 1:2] - x2t[1:2, :]
    d2c = x1[:, 2:3] - x2t[2:3, :]
    d = (d0 * d0 + d1c * d1c) + d2c * d2c

    inf = jnp.float32(jnp.inf)
    m1 = jnp.min(d, axis=1, keepdims=True)
    eq1 = d == m1
    d1 = jnp.where(eq1, inf, d)
    m2 = jnp.min(d1, axis=1, keepdims=True)
    eq2 = d1 == m2
    d2 = jnp.where(eq2, inf, d1)
    m3 = jnp.min(d2, axis=1, keepdims=True)
    eq3 = d2 == m3

    r = [1.0 / jnp.maximum(m, 1e-10) for m in (m1, m2, m3)]
    norm = (r[0] + r[1]) + r[2]

    iota = lax.broadcasted_iota(jnp.int32, (RB, N2), 1)
    base = b * N2
    for k, eq in enumerate((eq1, eq2, eq3)):
        idx_k = jnp.min(jnp.where(eq, iota, N2), axis=1, keepdims=True)
        gidx_ref[:, k:k + 1] = idx_k + base
        w_ref[:, k * 16:(k + 1) * 16] = jnp.broadcast_to(r[k] / norm,
                                                         (RB, 16))


def _sc_gather_body(gidx_hbm, wf_hbm, q2_hbm, out_hbm,
                    idx_all, w_all, rows0, rows1, out_v, sem0, sem1,
                    *, PPW, CHUNK, H, NC):
    wid = lax.axis_index("s") * NC + lax.axis_index("c")
    base = wid * PPW
    nchunks = PPW // CHUNK
    npairs = nchunks // 2

    pltpu.sync_copy(gidx_hbm.at[pl.ds(base * 3, PPW * 3)], idx_all)
    pltpu.sync_copy(wf_hbm.at[pl.ds(base * 48, PPW * 48)], w_all)

    def start_gather(ci, rows_v, sem):
        idx_sl = idx_all.at[pl.ds(ci * CHUNK * 3, CHUNK * 3)]
        return pltpu.async_copy(q2_hbm.at[idx_sl], rows_v, sem)

    def wait_gather(ci, rows_v, sem):
        idx_sl = idx_all.at[pl.ds(ci * CHUNK * 3, CHUNK * 3)]
        pltpu.make_async_copy(q2_hbm.at[idx_sl], rows_v, sem).wait()

    def compute_chunk(ci, rows_v):
        wbase3 = ci * CHUNK * 3

        def grp_body(g, c2):
            for u in range(4):
                p = g * 4 + u
                p3 = p * 3
                wb = (wbase3 + p3) * 16
                w0 = w_all[pl.ds(wb, 16)]
                w1 = w_all[pl.ds(wb + 16, 16)]
                w2 = w_all[pl.ds(wb + 32, 16)]
                for c in range(H // 16):
                    sl = pl.ds(c * 16, 16)
                    out_v[p, sl] = (w0 * rows_v[p3, sl]
                                    + w1 * rows_v[p3 + 1, sl]
                                    + w2 * rows_v[p3 + 2, sl])
            return c2

        lax.fori_loop(0, CHUNK // 4, grp_body, 0)
        pltpu.sync_copy(out_v, out_hbm.at[pl.ds(base + ci * CHUNK, CHUNK)])

    start_gather(0, rows0, sem0)

    def pair_body(q, carry):
        c0 = 2 * q
        start_gather(c0 + 1, rows1, sem1)
        wait_gather(c0, rows0, sem0)
        compute_chunk(c0, rows0)

        @pl.when(q + 1 < npairs)
        def _():
            start_gather(c0 + 2, rows0, sem0)

        wait_gather(c0 + 1, rows1, sem1)
        compute_chunk(c0 + 1, rows1)
        return carry

    lax.fori_loop(0, npairs, pair_body, 0)


def _stage2_body(interp_ref, p1_ref, W1_ref, b1_ref, W2_ref, b2_ref,
                 out_ref, *, C2):
    p1h = jnp.dot(p1_ref[...], W1_ref[C2:, :],
                  preferred_element_type=jnp.float32)
    h1 = jnp.maximum(interp_ref[...] + p1h + b1_ref[...], 0.0)
    h2 = jnp.dot(h1, W2_ref[...], preferred_element_type=jnp.float32)
    out_ref[...] = jnp.maximum(h2 + b2_ref[...], 0.0)


def kernel(xyz1, xyz2, points1, points2, W1, b1, W2, b2):
    B, N1, _ = xyz1.shape
    O = W2.shape[1]
    B2 = B // 4
    outs = [
        _half_pipeline(xyz1[h * B2:(h + 1) * B2],
                       xyz2[h * B2:(h + 1) * B2],
                       points1[h * B2:(h + 1) * B2],
                       points2[h * B2:(h + 1) * B2],
                       W1, b1, W2, b2)
        for h in range(4)
    ]
    return jnp.concatenate(outs, axis=0)


def _half_pipeline(xyz1, xyz2, points1, points2, W1, b1, W2, b2):
    B, N1, _ = xyz1.shape
    N2 = xyz2.shape[1]
    C1 = points1.shape[2]
    C2 = points2.shape[2]
    H = W1.shape[1]
    O = W2.shape[1]
    P = B * N1

    RB = 1024
    NB = N1 // RB

    x2t = jnp.transpose(xyz2, (0, 2, 1))
    b1r = b1.reshape(1, H)
    b2r = b2.reshape(1, O)

    s1 = functools.partial(_stage1_body, N2=N2, C2=C2, RB=RB)
    gidx, wf, q2 = pl.pallas_call(
        s1,
        grid=(B, NB),
        in_specs=[
            pl.BlockSpec((None, RB, 3), lambda b, j: (b, j, 0)),
            pl.BlockSpec((None, 3, N2), lambda b, j: (b, 0, 0)),
            pl.BlockSpec((None, N2, C2), lambda b, j: (b, 0, 0)),
            pl.BlockSpec((C1 + C2, H), lambda b, j: (0, 0)),
        ],
        out_specs=[
            pl.BlockSpec((None, RB, 3), lambda b, j: (b, j, 0)),
            pl.BlockSpec((None, RB, 48), lambda b, j: (b, j, 0)),
            pl.BlockSpec((None, N2, H), lambda b, j: (b, 0, 0)),
        ],
        out_shape=[
            jax.ShapeDtypeStruct((B, N1, 3), jnp.int32),
            jax.ShapeDtypeStruct((B, N1, 48), jnp.float32),
            jax.ShapeDtypeStruct((B, N2, H), jnp.float32),
        ],
    )(xyz1, x2t, points2, W1)

    gidxf = gidx.reshape(P * 3)
    wff = wf.reshape(P * 48)
    q2f = q2.reshape(B * N2, H)

    info = plsc.get_sparse_core_info()
    NC, NS = info.num_cores, info.num_subcores
    NW = NC * NS
    PPW = P // NW
    CHUNK = 32

    mesh = plsc.VectorSubcoreMesh(core_axis_name="c", subcore_axis_name="s")
    sc_body = functools.partial(_sc_gather_body, PPW=PPW, CHUNK=CHUNK,
                                H=H, NC=NC)
    interp = pl.kernel(
        sc_body,
        mesh=mesh,
        out_type=jax.ShapeDtypeStruct((P, H), jnp.float32),
        scratch_types=[
            pltpu.VMEM((PPW * 3,), jnp.int32),
            pltpu.VMEM((PPW * 48,), jnp.float32),
            pltpu.VMEM((CHUNK * 3, H), jnp.float32),
            pltpu.VMEM((CHUNK * 3, H), jnp.float32),
            pltpu.VMEM((CHUNK, H), jnp.float32),
            pltpu.SemaphoreType.DMA,
            pltpu.SemaphoreType.DMA,
        ],
    )(gidxf, wff, q2f)

    RB2 = 2048
    s2 = functools.partial(_stage2_body, C2=C2)
    out = pl.pallas_call(
        s2,
        grid=(P // RB2,),
        in_specs=[
            pl.BlockSpec((RB2, H), lambda i: (i, 0)),
            pl.BlockSpec((RB2, C1), lambda i: (i, 0)),
            pl.BlockSpec((C1 + C2, H), lambda i: (0, 0)),
            pl.BlockSpec((1, H), lambda i: (0, 0)),
            pl.BlockSpec((H, O), lambda i: (0, 0)),
            pl.BlockSpec((1, O), lambda i: (0, 0)),
        ],
        out_specs=pl.BlockSpec((RB2, O), lambda i: (i, 0)),
        out_shape=jax.ShapeDtypeStruct((P, O), jnp.float32),
    )(interp, points1.reshape(P, C1), W1, b1r, W2, b2r)
    return out.reshape(B, N1, O)
